# Initial kernel scaffold; baseline (speedup 1.0000x reference)
import jax
import jax.numpy as jnp
from jax.experimental import pallas as pl


def kernel(pos_embs, neg_embs, neut_embs, keys, W1, b1, W2, b2, W3, b3, W4, b4):
    raise NotImplementedError("write your pallas kernel here")


# speedup vs baseline: 5.2409x; 5.2409x over previous
import functools

import jax
import jax.numpy as jnp
from jax import lax
from jax.experimental import pallas as pl
from jax.experimental.pallas import tpu as pltpu
from jax.experimental.pallas import tpu_sc as plsc

B = 256
D = 768
K = 100000
TOPK = 50
CHUNK = 128
NBLK = 800
KPAD = NBLK * CHUNK
KB = 4096
NSTEPS = KPAD // KB
CPS = KB // CHUNK
NCAND = TOPK * CHUNK
NEG = -3.0e38
BIGI = 2**30

_NC, _NS = 2, 16
NW = _NC * _NS


def _mm_t(a, b):
    return lax.dot_general(
        a.astype(jnp.bfloat16), b.astype(jnp.bfloat16),
        (((1,), (1,)), ((), ())), preferred_element_type=jnp.float32)


def _mm(a, b):
    return lax.dot_general(
        a.astype(jnp.bfloat16), b.astype(jnp.bfloat16),
        (((1,), (0,)), ((), ())), preferred_element_type=jnp.float32)


def _norm_rows(x):
    n = jnp.sqrt(jnp.sum(x * x, axis=1, keepdims=True))
    return x / jnp.clip(n, 1e-12, None)


def _prep_body(pos_ref, neg_ref, neut_ref, x_ref, v_ref):
    def mean_norm(ref, n):
        s = ref[:, 0, :]
        for i in range(1, n):
            s = s + ref[:, i, :]
        return _norm_rows(s * jnp.float32(1.0 / n))

    pos = mean_norm(pos_ref, 9)
    neg = mean_norm(neg_ref, 9)
    neut = mean_norm(neut_ref, 6)
    x_ref[...] = jnp.concatenate([neg, neut, pos], axis=1)
    v_ref[...] = neg * 0.5 + neut * 0.5 - pos


def _prep(pos, neg, neut):
    return pl.pallas_call(
        _prep_body,
        out_shape=[jax.ShapeDtypeStruct((B, 3 * D), jnp.float32),
                   jax.ShapeDtypeStruct((B, D), jnp.float32)],
    )(pos, neg, neut)


def _tower_body(x_ref, w1, b1, w2, b2, w3, b3, w4, b4, mo_ref):
    h = jnp.maximum(_mm(x_ref[...], w1[...]) + b1[...], 0.0)
    h = jnp.maximum(_mm(h, w2[...]) + b2[...], 0.0)
    h = jnp.maximum(_mm(h, w3[...]) + b3[...], 0.0)
    h = _mm(h, w4[...]) + b4[...]
    mo_ref[...] = _norm_rows(h)


def _tower(x, W1, b1, W2, b2, W3, b3, W4, b4):
    return pl.pallas_call(
        _tower_body,
        out_shape=jax.ShapeDtypeStruct((B, D), jnp.float32),
    )(x, W1, b1.reshape(1, -1), W2, b2.reshape(1, -1),
      W3, b3.reshape(1, -1), W4, b4.reshape(1, -1))


def _scores_body(mo_ref, keys_ref, s_ref, bmax_ref):
    j = pl.program_id(0)
    s = _mm_t(mo_ref[...], keys_ref[...])
    col = j * KB + lax.broadcasted_iota(jnp.int32, (B, KB), 1)
    s = jnp.where(col < K, s, NEG)
    s_ref[...] = s
    parts = [jnp.max(s[:, c * CHUNK:(c + 1) * CHUNK], axis=1, keepdims=True)
             for c in range(CPS)]
    bmax_ref[0] = jnp.concatenate(parts, axis=1)


def _scores(mo, keys):
    return pl.pallas_call(
        _scores_body,
        grid=(NSTEPS,),
        in_specs=[pl.BlockSpec((B, D), lambda j: (0, 0)),
                  pl.BlockSpec((KB, D), lambda j: (j, 0))],
        out_specs=[pl.BlockSpec((B, KB), lambda j: (0, j)),
                   pl.BlockSpec((1, B, CPS), lambda j: (j, 0, 0))],
        out_shape=[jax.ShapeDtypeStruct((B, KPAD), jnp.float32),
                   jax.ShapeDtypeStruct((NSTEPS, B, CPS), jnp.float32)],
    )(mo, keys)


def _blocktop_body(bmax_ref, rid_ref, x_ref, acc_ref):
    x_ref[...] = bmax_ref[...]
    acc_ref[...] = jnp.zeros((B, 64), jnp.int32)
    colv = lax.broadcasted_iota(jnp.int32, (B, NBLK), 1)
    lane64 = lax.broadcasted_iota(jnp.int32, (B, 64), 1)

    def step(i, _):
        x = x_ref[...]
        m = jnp.max(x, axis=1, keepdims=True)
        sel = jnp.min(jnp.where(x == m, colv, BIGI), axis=1, keepdims=True)
        acc_ref[...] = jnp.where(lane64 == i, sel, acc_ref[...])
        x_ref[...] = jnp.where(colv == sel, NEG, x)
        return 0

    lax.fori_loop(0, TOPK, step, 0)
    row = lax.broadcasted_iota(jnp.int32, (B, 64), 0)
    rid_ref[...] = (acc_ref[...] + row * NBLK)[:, :TOPK]


def _blocktop(bmax):
    return pl.pallas_call(
        _blocktop_body,
        out_shape=jax.ShapeDtypeStruct((B, TOPK), jnp.int32),
        scratch_shapes=[pltpu.VMEM((B, NBLK), jnp.float32),
                        pltpu.VMEM((B, 64), jnp.int32)],
    )(bmax)


def _sc_gather_rows(table, idx, rows_per_chunk):
    n, d = idx.shape[0], table.shape[1]
    b_per_w = n // NW
    nchunks = b_per_w // rows_per_chunk
    assert nchunks * rows_per_chunk == b_per_w and rows_per_chunk % 8 == 0
    mesh = plsc.VectorSubcoreMesh(core_axis_name="c", subcore_axis_name="s")

    @functools.partial(
        pl.kernel, mesh=mesh,
        out_type=jax.ShapeDtypeStruct((n, d), jnp.float32),
        scratch_types=[pltpu.VMEM((rows_per_chunk,), jnp.int32),
                       pltpu.VMEM((rows_per_chunk, d), jnp.float32),
                       pltpu.SemaphoreType.DMA],
    )
    def k(table_hbm, idx_hbm, out_hbm, idx_v, rows_v, sem):
        wid = lax.axis_index("s") * _NC + lax.axis_index("c")
        for t in range(nchunks):
            base = wid * b_per_w + t * rows_per_chunk
            pltpu.sync_copy(idx_hbm.at[pl.ds(base, rows_per_chunk)], idx_v)
            pltpu.async_copy(table_hbm.at[idx_v], rows_v, sem).wait()
            pltpu.sync_copy(rows_v, out_hbm.at[pl.ds(base, rows_per_chunk)])

    return k(table, idx)


def _candtop_body(cand_ref, rid_ref, idx_ref, x_ref, acc_ref):
    x_ref[...] = cand_ref[...]
    acc_ref[...] = jnp.zeros((B, 64), jnp.int32)
    rid = rid_ref[...]
    row = lax.broadcasted_iota(jnp.int32, (B, TOPK), 0)
    blk = rid - row * NBLK
    off = lax.broadcasted_iota(jnp.int32, (B, CHUNK), 1)
    gmap = jnp.concatenate(
        [blk[:, i:i + 1] * CHUNK + off for i in range(TOPK)], axis=1)
    lane64 = lax.broadcasted_iota(jnp.int32, (B, 64), 1)

    def step(i, _):
        x = x_ref[...]
        m = jnp.max(x, axis=1, keepdims=True)
        sel = jnp.min(jnp.where(x == m, gmap, BIGI), axis=1, keepdims=True)
        acc_ref[...] = jnp.where(lane64 == i, sel, acc_ref[...])
        x_ref[...] = jnp.where(gmap == sel, NEG, x)
        return 0

    lax.fori_loop(0, TOPK, step, 0)
    idx_ref[...] = acc_ref[...][:, :TOPK]


def _candtop(cand, rid):
    return pl.pallas_call(
        _candtop_body,
        out_shape=jax.ShapeDtypeStruct((B, TOPK), jnp.int32),
        scratch_shapes=[pltpu.VMEM((B, NCAND), jnp.float32),
                        pltpu.VMEM((B, 64), jnp.int32)],
    )(cand, rid)


def _select_body(w_ref, v_ref, omin_ref, omax_ref):
    v = v_ref[...]
    rows = w_ref.shape[0]
    lparts = [jnp.sum(w_ref[:, i, :] * v, axis=1, keepdims=True)
              for i in range(TOPK)]
    l = jnp.concatenate(lparts, axis=1)
    i50 = lax.broadcasted_iota(jnp.int32, (rows, TOPK), 1)

    def pick(target):
        am = jnp.min(jnp.where(l == target, i50, BIGI), axis=1, keepdims=True)
        oh = (i50 == am).astype(jnp.float32)
        out = jnp.zeros((rows, D), jnp.float32)
        for i in range(TOPK):
            out = out + oh[:, i:i + 1] * w_ref[:, i, :]
        return out

    omin_ref[...] = pick(jnp.min(l, axis=1, keepdims=True))
    omax_ref[...] = pick(jnp.max(l, axis=1, keepdims=True))


def _select(wemb, v):
    rb = 64
    return pl.pallas_call(
        _select_body,
        grid=(B // rb,),
        in_specs=[pl.BlockSpec((rb, TOPK, D), lambda j: (j, 0, 0)),
                  pl.BlockSpec((rb, D), lambda j: (j, 0))],
        out_specs=[pl.BlockSpec((rb, D), lambda j: (j, 0)),
                   pl.BlockSpec((rb, D), lambda j: (j, 0))],
        out_shape=[jax.ShapeDtypeStruct((B, D), jnp.float32),
                   jax.ShapeDtypeStruct((B, D), jnp.float32)],
    )(wemb, v)


def kernel(pos_embs, neg_embs, neut_embs, keys, W1, b1, W2, b2, W3, b3, W4, b4):
    x, v = _prep(pos_embs, neg_embs, neut_embs)
    mo = _tower(x, W1, b1, W2, b2, W3, b3, W4, b4)
    s, bmax3 = _scores(mo, keys)
    bmax = bmax3.transpose(1, 0, 2).reshape(B, NBLK)
    rid = _blocktop(bmax)
    cand = _sc_gather_rows(s.reshape(B * NBLK, CHUNK), rid.reshape(-1), 400)
    idx50 = _candtop(cand.reshape(B, NCAND), rid)
    wemb = _sc_gather_rows(keys, idx50.reshape(-1), 80)
    omin, omax = _select(wemb.reshape(B, TOPK, D), v)
    return (mo, omax, omin)
